# Initial kernel scaffold; baseline (speedup 1.0000x reference)
import jax
import jax.numpy as jnp
from jax.experimental import pallas as pl


def kernel(x, x_contrast, stoken_size):
    raise NotImplementedError("write your pallas kernel here")


# speedup vs baseline: 36.7741x; 36.7741x over previous
import jax
import jax.numpy as jnp
from jax import lax
from jax.experimental import pallas as pl
from jax.experimental.pallas import tpu as pltpu

_ST = 16
_NS = 14
_S = _NS * _NS
_C1 = 96
_W = 224
_STRIP = _ST * _W
_NSTRIP = _W // _ST

_MM = dict(preferred_element_type=jnp.float32, precision=lax.Precision.HIGHEST)


def _pool_kernel(mpool_ref, xc_ref, xk_ref, cent_ref):
    j = pl.program_id(1)
    pixc = xc_ref[0]
    pixk = xk_ref[0]
    mp = mpool_ref[...]
    cellc = jnp.dot(pixc, mp, **_MM) * (1.0 / 256.0)
    cellk = jnp.dot(pixk, mp, **_MM) * (1.0 / 256.0)
    trow = lax.broadcasted_iota(jnp.int32, (_NS, _S), 0)
    scol = lax.broadcasted_iota(jnp.int32, (_NS, _S), 1)
    ej = (scol == trow + j * _NS).astype(jnp.float32)
    fc = jnp.dot(cellc, ej, **_MM)
    fk = jnp.dot(cellk, ej, **_MM)

    @pl.when(j == 0)
    def _init():
        cent_ref[0, :_C1, :] = fc
        cent_ref[0, _C1:, :] = fk

    @pl.when(j != 0)
    def _acc():
        cent_ref[0, :_C1, :] += fc
        cent_ref[0, _C1:, :] += fk


def _affinities(syc_ref, m2_ref, pixc, pixk, centc, centk, j):
    d = lax.dot_general(centc, pixc, (((0,), (0,)), ((), ())), **_MM)
    d += lax.dot_general(centk * 10.0, pixk, (((0,), (0,)), ((), ())), **_MM)
    c2 = (jnp.sum(centc * centc, axis=0, keepdims=True)
          + 10.0 * jnp.sum(centk * centk, axis=0, keepdims=True))
    dist = lax.dot_general(c2, jnp.ones((1, _STRIP), jnp.float32),
                           (((0,), (0,)), ((), ())), **_MM) - 2.0 * d
    band = jnp.abs(syc_ref[...] - j) <= 1
    maskb = jnp.logical_and(m2_ref[...] > 0.5, band)
    mmin = jnp.min(jnp.where(maskb, dist, 1e30), axis=0, keepdims=True)
    z = jnp.where(maskb, mmin - dist, -1e9)
    e = jnp.exp(z)
    return e / jnp.sum(e, axis=0, keepdims=True)


def _iter1_kernel(syc_ref, m2_ref, xc_ref, xk_ref, cent_ref,
                  num_ref, den_ref):
    j = pl.program_id(1)
    pixc = xc_ref[0]
    pixk = xk_ref[0]
    aff = _affinities(syc_ref, m2_ref, pixc, pixk,
                      cent_ref[0, :_C1, :], cent_ref[0, _C1:, :], j)
    afft = aff.T
    nc = jnp.dot(pixc, afft, **_MM)
    nk = jnp.dot(pixk, afft, **_MM)
    dden = jnp.sum(afft, axis=0, keepdims=True)

    @pl.when(j == 0)
    def _init():
        num_ref[0, :_C1, :] = nc
        num_ref[0, _C1:, :] = nk
        den_ref[0] = dden

    @pl.when(j != 0)
    def _acc():
        num_ref[0, :_C1, :] += nc
        num_ref[0, _C1:, :] += nk
        den_ref[0] += dden


def _iter2_kernel(syc_ref, m2_ref, xc_ref, xk_ref, num_ref, den_ref,
                  out_ref):
    j = pl.program_id(1)
    r = 1.0 / (den_ref[0] + 1e-16)
    centc = num_ref[0, :_C1, :] * r
    centk = num_ref[0, _C1:, :] * r
    out_ref[0] = _affinities(syc_ref, m2_ref, xc_ref[0], xk_ref[0],
                             centc, centk, j)


def kernel(x, x_contrast, stoken_size):
    del stoken_size
    b = x.shape[0]
    xr = x.reshape(b, _C1, _W * _W)
    xkr = x_contrast.reshape(b, _C1, _W * _W)

    sp = jnp.arange(_S, dtype=jnp.int32)
    syc = (sp // _NS)[:, None]
    icol = ((jnp.arange(_STRIP, dtype=jnp.int32) % _W) // _ST)[None, :]
    m2 = (jnp.abs((sp % _NS)[:, None] - icol) <= 1).astype(jnp.float32)
    mpool = (icol[0][:, None] == jnp.arange(_NS, dtype=jnp.int32)[None, :]
             ).astype(jnp.float32)

    grid = (b, _NSTRIP)
    strip_spec = pl.BlockSpec((1, _C1, _STRIP), lambda bb, j: (bb, 0, j))
    whole = lambda shape: pl.BlockSpec(shape, lambda bb, j: (0,) * len(shape))
    params = pltpu.CompilerParams(
        dimension_semantics=("arbitrary", "arbitrary"))

    cent = pl.pallas_call(
        _pool_kernel,
        grid=grid,
        in_specs=[whole((_STRIP, _NS)), strip_spec, strip_spec],
        out_specs=pl.BlockSpec((1, 2 * _C1, _S), lambda bb, j: (bb, 0, 0)),
        out_shape=jax.ShapeDtypeStruct((b, 2 * _C1, _S), jnp.float32),
        compiler_params=params,
    )(mpool, xr, xkr)

    num, den = pl.pallas_call(
        _iter1_kernel,
        grid=grid,
        in_specs=[whole((_S, 1)), whole((_S, _STRIP)), strip_spec, strip_spec,
                  pl.BlockSpec((1, 2 * _C1, _S), lambda bb, j: (bb, 0, 0))],
        out_specs=[pl.BlockSpec((1, 2 * _C1, _S), lambda bb, j: (bb, 0, 0)),
                   pl.BlockSpec((1, 1, _S), lambda bb, j: (bb, 0, 0))],
        out_shape=[jax.ShapeDtypeStruct((b, 2 * _C1, _S), jnp.float32),
                   jax.ShapeDtypeStruct((b, 1, _S), jnp.float32)],
        compiler_params=params,
    )(syc, m2, xr, xkr, cent)

    out = pl.pallas_call(
        _iter2_kernel,
        grid=grid,
        in_specs=[whole((_S, 1)), whole((_S, _STRIP)), strip_spec, strip_spec,
                  pl.BlockSpec((1, 2 * _C1, _S), lambda bb, j: (bb, 0, 0)),
                  pl.BlockSpec((1, 1, _S), lambda bb, j: (bb, 0, 0))],
        out_specs=pl.BlockSpec((1, _S, _STRIP), lambda bb, j: (bb, 0, j)),
        out_shape=jax.ShapeDtypeStruct((b, _S, _W * _W), jnp.float32),
        compiler_params=params,
    )(syc, m2, xr, xkr, num, den)

    return out, _S


# speedup vs baseline: 54.0589x; 1.4700x over previous
import jax
import jax.numpy as jnp
from jax import lax
from jax.experimental import pallas as pl
from jax.experimental.pallas import tpu as pltpu

_ST = 16
_NS = 14
_S = _NS * _NS
_C1 = 96
_W = 224
_STRIP = _ST * _W
_NSTRIP = _W // _ST

_MM = dict(preferred_element_type=jnp.float32)


def _split(a):
    ah = a.astype(jnp.bfloat16)
    al = (a - ah.astype(jnp.float32)).astype(jnp.bfloat16)
    return ah, al


def _dot3(a, b, dims):
    ah, al = _split(a)
    bh, bl = _split(b)
    d = lax.dot_general(ah, bh, (dims, ((), ())), **_MM)
    d += lax.dot_general(ah, bl, (dims, ((), ())), **_MM)
    d += lax.dot_general(al, bh, (dims, ((), ())), **_MM)
    return d


def _dot3p(ah, al, bh, bl, dims):
    d = lax.dot_general(ah, bh, (dims, ((), ())), **_MM)
    d += lax.dot_general(ah, bl, (dims, ((), ())), **_MM)
    d += lax.dot_general(al, bh, (dims, ((), ())), **_MM)
    return d


def _dot_e(a, b_exact, dims):
    ah, al = _split(a)
    be = b_exact.astype(jnp.bfloat16)
    d = lax.dot_general(ah, be, (dims, ((), ())), **_MM)
    d += lax.dot_general(al, be, (dims, ((), ())), **_MM)
    return d


def _dot_e2(a_exact, b, dims):
    ae = a_exact.astype(jnp.bfloat16)
    bh, bl = _split(b)
    d = lax.dot_general(ae, bh, (dims, ((), ())), **_MM)
    d += lax.dot_general(ae, bl, (dims, ((), ())), **_MM)
    return d


_WIN = 48


def _window(j):
    q = jnp.clip(j - 1, 0, 11)
    w0 = q * _NS
    s_iota = lax.broadcasted_iota(jnp.int32, (_S, _WIN), 0)
    r_iota = lax.broadcasted_iota(jnp.int32, (_S, _WIN), 1)
    pjt = (s_iota == r_iota + w0).astype(jnp.float32)
    s_iota2 = lax.broadcasted_iota(jnp.int32, (_WIN, _S), 1)
    r_iota2 = lax.broadcasted_iota(jnp.int32, (_WIN, _S), 0)
    pj = (s_iota2 == r_iota2 + w0).astype(jnp.float32)
    rcol = lax.broadcasted_iota(jnp.int32, (_WIN, 1), 0)
    rband = (rcol >= _NS).astype(jnp.int32) + (rcol >= 2 * _NS).astype(
        jnp.int32) + (rcol >= 3 * _NS).astype(jnp.int32)
    band_ok = (jnp.abs(rband + q - j) <= 1) & (rcol + w0 <= _S - 1)
    rmod = rcol - rband * _NS
    return pjt, pj, band_ok, rmod


def _affinities_w(icol_ref, pixs, centwc, centwk, band_ok, rmod):
    (pch, pcl), (pkh, pkl) = pixs
    d = _dot3p(*_split(centwc), pch, pcl, ((0,), (0,)))
    d += _dot3p(*_split(centwk * 10.0), pkh, pkl, ((0,), (0,)))
    c2 = (jnp.sum(centwc * centwc, axis=0, keepdims=True)
          + 10.0 * jnp.sum(centwk * centwk, axis=0, keepdims=True))
    dist = _dot_e(c2, jnp.ones((1, _STRIP), jnp.float32),
                  ((0,), (0,))) - 2.0 * d
    maskb = (jnp.abs(rmod - icol_ref[...]) <= 1) & band_ok
    mmin = jnp.min(jnp.where(maskb, dist, 1e30), axis=0, keepdims=True)
    z = jnp.where(maskb, mmin - dist, -1e9)
    e = jnp.exp(z)
    return e / jnp.sum(e, axis=0, keepdims=True)


def _pool_kernel(mpool_ref, xc_ref, xk_ref, cent_ref):
    j = pl.program_id(1)
    rowc = jnp.sum(xc_ref[0], axis=1)
    rowk = jnp.sum(xk_ref[0], axis=1)
    mp = mpool_ref[...]
    cellc = _dot_e(rowc, mp, ((1,), (0,))) * (1.0 / 256.0)
    cellk = _dot_e(rowk, mp, ((1,), (0,))) * (1.0 / 256.0)
    trow = lax.broadcasted_iota(jnp.int32, (_NS, _S), 0)
    scol = lax.broadcasted_iota(jnp.int32, (_NS, _S), 1)
    ej = (scol == trow + j * _NS).astype(jnp.float32)
    fc = _dot_e(cellc, ej, ((1,), (0,)))
    fk = _dot_e(cellk, ej, ((1,), (0,)))

    @pl.when(j == 0)
    def _init():
        cent_ref[0, :_C1, :] = fc
        cent_ref[0, _C1:, :] = fk

    @pl.when(j != 0)
    def _acc():
        cent_ref[0, :_C1, :] += fc
        cent_ref[0, _C1:, :] += fk


def _iter1_kernel(icol_ref, xc_ref, xk_ref, cent_ref, num_ref, den_ref):
    j = pl.program_id(1)
    pjt, pj, band_ok, rmod = _window(j)
    pixs = (_split(xc_ref[0]), _split(xk_ref[0]))
    centwc = _dot_e(cent_ref[0, :_C1, :], pjt, ((1,), (0,)))
    centwk = _dot_e(cent_ref[0, _C1:, :], pjt, ((1,), (0,)))
    aff = _affinities_w(icol_ref, pixs, centwc, centwk, band_ok, rmod)
    ah, al = _split(aff)
    afth, aftl = ah.T, al.T
    (pch, pcl), (pkh, pkl) = pixs
    ncw = _dot3p(pch, pcl, afth, aftl, ((1,), (0,)))
    nkw = _dot3p(pkh, pkl, afth, aftl, ((1,), (0,)))
    ddenw = jnp.sum(aff, axis=1, keepdims=True)
    nc = _dot_e(ncw, pj, ((1,), (0,)))
    nk = _dot_e(nkw, pj, ((1,), (0,)))
    dden = _dot_e2(pjt, ddenw, ((1,), (0,)))

    @pl.when(j == 0)
    def _init():
        num_ref[0, :_C1, :] = nc
        num_ref[0, _C1:, :] = nk
        den_ref[0] = dden

    @pl.when(j != 0)
    def _acc():
        num_ref[0, :_C1, :] += nc
        num_ref[0, _C1:, :] += nk
        den_ref[0] += dden


def _iter2_kernel(icol_ref, xc_ref, xk_ref, num_ref, den_ref, out_ref):
    j = pl.program_id(1)
    pjt, pj, band_ok, rmod = _window(j)
    numwc = _dot_e(num_ref[0, :_C1, :], pjt, ((1,), (0,)))
    numwk = _dot_e(num_ref[0, _C1:, :], pjt, ((1,), (0,)))
    denw = _dot_e2(pj, den_ref[0], ((1,), (0,)))
    r = (1.0 / (denw + 1e-16)).T
    pixs = (_split(xc_ref[0]), _split(xk_ref[0]))
    aff = _affinities_w(icol_ref, pixs, numwc * r, numwk * r, band_ok, rmod)
    out_ref[0, 0] = _dot_e2(pjt, aff, ((1,), (0,)))


def kernel(x, x_contrast, stoken_size):
    del stoken_size
    b = x.shape[0]
    xr = x.reshape(b, _C1, _W * _W)
    xkr = x_contrast.reshape(b, _C1, _W * _W)

    icol = ((jnp.arange(_STRIP, dtype=jnp.int32) % _W) // _ST)[None, :]
    mpool = ((jnp.arange(_W, dtype=jnp.int32) // _ST)[:, None]
             == jnp.arange(_NS, dtype=jnp.int32)[None, :]
             ).astype(jnp.float32)

    grid = (b, _NSTRIP)
    strip_spec = pl.BlockSpec((1, _C1, _STRIP), lambda bb, j: (bb, 0, j))
    whole = lambda shape: pl.BlockSpec(shape, lambda bb, j: (0,) * len(shape))
    params = pltpu.CompilerParams(
        dimension_semantics=("arbitrary", "arbitrary"))

    strip4_spec = pl.BlockSpec((1, _C1, _ST, _W), lambda bb, j: (bb, 0, j, 0))
    cent = pl.pallas_call(
        _pool_kernel,
        grid=grid,
        in_specs=[whole((_W, _NS)), strip4_spec, strip4_spec],
        out_specs=pl.BlockSpec((1, 2 * _C1, _S), lambda bb, j: (bb, 0, 0)),
        out_shape=jax.ShapeDtypeStruct((b, 2 * _C1, _S), jnp.float32),
        compiler_params=params,
    )(mpool, x, x_contrast)

    num, den = pl.pallas_call(
        _iter1_kernel,
        grid=grid,
        in_specs=[whole((1, _STRIP)), strip_spec, strip_spec,
                  pl.BlockSpec((1, 2 * _C1, _S), lambda bb, j: (bb, 0, 0))],
        out_specs=[pl.BlockSpec((1, 2 * _C1, _S), lambda bb, j: (bb, 0, 0)),
                   pl.BlockSpec((1, _S, 1), lambda bb, j: (bb, 0, 0))],
        out_shape=[jax.ShapeDtypeStruct((b, 2 * _C1, _S), jnp.float32),
                   jax.ShapeDtypeStruct((b, _S, 1), jnp.float32)],
        compiler_params=params,
    )(icol, xr, xkr, cent)

    out = pl.pallas_call(
        _iter2_kernel,
        grid=grid,
        in_specs=[whole((1, _STRIP)), strip_spec, strip_spec,
                  pl.BlockSpec((1, 2 * _C1, _S), lambda bb, j: (bb, 0, 0)),
                  pl.BlockSpec((1, _S, 1), lambda bb, j: (bb, 0, 0))],
        out_specs=pl.BlockSpec((1, 1, _S, _STRIP), lambda bb, j: (bb, j, 0, 0)),
        out_shape=jax.ShapeDtypeStruct((b, _NSTRIP, _S, _STRIP), jnp.float32),
        compiler_params=params,
    )(icol, xr, xkr, num, den)

    out = out.transpose(0, 2, 1, 3).reshape(b, _S, _W * _W)
    return out, _S


# speedup vs baseline: 66.1907x; 1.2244x over previous
import jax
import jax.numpy as jnp
from jax import lax
from jax.experimental import pallas as pl
from jax.experimental.pallas import tpu as pltpu

_ST = 16
_NS = 14
_S = _NS * _NS
_C1 = 96
_W = 224
_STRIP = _ST * _W
_NSTRIP = _W // _ST

_MM = dict(preferred_element_type=jnp.float32)


def _split(a):
    ah = a.astype(jnp.bfloat16)
    al = (a - ah.astype(jnp.float32)).astype(jnp.bfloat16)
    return ah, al


def _dot3(a, b, dims):
    ah, al = _split(a)
    bh, bl = _split(b)
    d = lax.dot_general(ah, bh, (dims, ((), ())), **_MM)
    d += lax.dot_general(ah, bl, (dims, ((), ())), **_MM)
    d += lax.dot_general(al, bh, (dims, ((), ())), **_MM)
    return d


def _dot3p(ah, al, bh, bl, dims):
    d = lax.dot_general(ah, bh, (dims, ((), ())), **_MM)
    d += lax.dot_general(ah, bl, (dims, ((), ())), **_MM)
    d += lax.dot_general(al, bh, (dims, ((), ())), **_MM)
    return d


def _dot_e(a, b_exact, dims):
    ah, al = _split(a)
    be = b_exact.astype(jnp.bfloat16)
    d = lax.dot_general(ah, be, (dims, ((), ())), **_MM)
    d += lax.dot_general(al, be, (dims, ((), ())), **_MM)
    return d


def _dot_e2(a_exact, b, dims):
    ae = a_exact.astype(jnp.bfloat16)
    bh, bl = _split(b)
    d = lax.dot_general(ae, bh, (dims, ((), ())), **_MM)
    d += lax.dot_general(ae, bl, (dims, ((), ())), **_MM)
    return d


_WIN = 48


def _window(j):
    q = jnp.clip(j - 1, 0, 11)
    w0 = q * _NS
    s_iota = lax.broadcasted_iota(jnp.int32, (_S, _WIN), 0)
    r_iota = lax.broadcasted_iota(jnp.int32, (_S, _WIN), 1)
    pjt = (s_iota == r_iota + w0).astype(jnp.float32)
    s_iota2 = lax.broadcasted_iota(jnp.int32, (_WIN, _S), 1)
    r_iota2 = lax.broadcasted_iota(jnp.int32, (_WIN, _S), 0)
    pj = (s_iota2 == r_iota2 + w0).astype(jnp.float32)
    rcol = lax.broadcasted_iota(jnp.int32, (_WIN, 1), 0)
    rband = (rcol >= _NS).astype(jnp.int32) + (rcol >= 2 * _NS).astype(
        jnp.int32) + (rcol >= 3 * _NS).astype(jnp.int32)
    band_ok = (jnp.abs(rband + q - j) <= 1) & (rcol + w0 <= _S - 1)
    rmod = rcol - rband * _NS
    return pjt, pj, band_ok, rmod


def _affinities_w(icol_ref, pixs, centwc, centwk, band_ok, rmod):
    (pch, pcl), (pkh, pkl) = pixs
    d = _dot3p(*_split(centwc), pch, pcl, ((0,), (0,)))
    d += _dot3p(*_split(centwk * 10.0), pkh, pkl, ((0,), (0,)))
    c2 = (jnp.sum(centwc * centwc, axis=0, keepdims=True)
          + 10.0 * jnp.sum(centwk * centwk, axis=0, keepdims=True))
    dist = _dot_e(c2, jnp.ones((1, _STRIP), jnp.float32),
                  ((0,), (0,))) - 2.0 * d
    maskb = (jnp.abs(rmod - icol_ref[...]) <= 1) & band_ok
    mmin = jnp.min(jnp.where(maskb, dist, 1e30), axis=0, keepdims=True)
    z = jnp.where(maskb, mmin - dist, -1e9)
    e = jnp.exp(z)
    return e / jnp.sum(e, axis=0, keepdims=True)


def _pool_kernel(mpool_ref, xc_ref, xk_ref, cent_ref):
    j = pl.program_id(1)
    rowc = jnp.sum(xc_ref[0], axis=1)
    rowk = jnp.sum(xk_ref[0], axis=1)
    mp = mpool_ref[...]
    cellc = _dot_e(rowc, mp, ((1,), (0,))) * (1.0 / 256.0)
    cellk = _dot_e(rowk, mp, ((1,), (0,))) * (1.0 / 256.0)
    trow = lax.broadcasted_iota(jnp.int32, (_NS, _S), 0)
    scol = lax.broadcasted_iota(jnp.int32, (_NS, _S), 1)
    ej = (scol == trow + j * _NS).astype(jnp.float32)
    fc = _dot_e(cellc, ej, ((1,), (0,)))
    fk = _dot_e(cellk, ej, ((1,), (0,)))

    @pl.when(j == 0)
    def _init():
        cent_ref[0, :_C1, :] = fc
        cent_ref[0, _C1:, :] = fk

    @pl.when(j != 0)
    def _acc():
        cent_ref[0, :_C1, :] += fc
        cent_ref[0, _C1:, :] += fk


def _iter1_kernel(icol_ref, xc_ref, xk_ref, cent_ref, num_ref, den_ref):
    j = pl.program_id(1)
    pjt, pj, band_ok, rmod = _window(j)
    pixs = (_split(xc_ref[0]), _split(xk_ref[0]))
    centwc = _dot_e(cent_ref[0, :_C1, :], pjt, ((1,), (0,)))
    centwk = _dot_e(cent_ref[0, _C1:, :], pjt, ((1,), (0,)))
    aff = _affinities_w(icol_ref, pixs, centwc, centwk, band_ok, rmod)
    ah, al = _split(aff)
    afth, aftl = ah.T, al.T
    (pch, pcl), (pkh, pkl) = pixs
    ncw = _dot3p(pch, pcl, afth, aftl, ((1,), (0,)))
    nkw = _dot3p(pkh, pkl, afth, aftl, ((1,), (0,)))
    ddenw = jnp.sum(aff, axis=1, keepdims=True)
    nc = _dot_e(ncw, pj, ((1,), (0,)))
    nk = _dot_e(nkw, pj, ((1,), (0,)))
    dden = _dot_e2(pjt, ddenw, ((1,), (0,)))

    @pl.when(j == 0)
    def _init():
        num_ref[0, :_C1, :] = nc
        num_ref[0, _C1:, :] = nk
        den_ref[0] = dden

    @pl.when(j != 0)
    def _acc():
        num_ref[0, :_C1, :] += nc
        num_ref[0, _C1:, :] += nk
        den_ref[0] += dden


def _iter2_kernel(icol_ref, xc_ref, xk_ref, num_ref, den_ref, out_ref,
                  stash_ref):
    j = pl.program_id(1)
    pjt, pj, band_ok, rmod = _window(j)
    numwc = _dot_e(num_ref[0, :_C1, :], pjt, ((1,), (0,)))
    numwk = _dot_e(num_ref[0, _C1:, :], pjt, ((1,), (0,)))
    denw = _dot_e2(pj, den_ref[0], ((1,), (0,)))
    r = (1.0 / (denw + 1e-16)).T
    pixs = (_split(xc_ref[0]), _split(xk_ref[0]))
    aff = _affinities_w(icol_ref, pixs, numwc * r, numwk * r, band_ok, rmod)
    full = _dot_e2(pjt, aff, ((1,), (0,)))

    @pl.when(j % 2 == 0)
    def _stash():
        stash_ref[...] = full

    @pl.when(j % 2 == 1)
    def _emit():
        out_ref[0, :, :_STRIP] = stash_ref[...]
        out_ref[0, :, _STRIP:] = full


def kernel(x, x_contrast, stoken_size):
    del stoken_size
    b = x.shape[0]
    xr = x.reshape(b, _C1, _W * _W)
    xkr = x_contrast.reshape(b, _C1, _W * _W)

    icol = ((jnp.arange(_STRIP, dtype=jnp.int32) % _W) // _ST)[None, :]
    mpool = ((jnp.arange(_W, dtype=jnp.int32) // _ST)[:, None]
             == jnp.arange(_NS, dtype=jnp.int32)[None, :]
             ).astype(jnp.float32)

    grid = (b, _NSTRIP)
    strip_spec = pl.BlockSpec((1, _C1, _STRIP), lambda bb, j: (bb, 0, j))
    whole = lambda shape: pl.BlockSpec(shape, lambda bb, j: (0,) * len(shape))
    params = pltpu.CompilerParams(
        dimension_semantics=("arbitrary", "arbitrary"))

    strip4_spec = pl.BlockSpec((1, _C1, _ST, _W), lambda bb, j: (bb, 0, j, 0))
    cent = pl.pallas_call(
        _pool_kernel,
        grid=grid,
        in_specs=[whole((_W, _NS)), strip4_spec, strip4_spec],
        out_specs=pl.BlockSpec((1, 2 * _C1, _S), lambda bb, j: (bb, 0, 0)),
        out_shape=jax.ShapeDtypeStruct((b, 2 * _C1, _S), jnp.float32),
        compiler_params=params,
    )(mpool, x, x_contrast)

    num, den = pl.pallas_call(
        _iter1_kernel,
        grid=grid,
        in_specs=[whole((1, _STRIP)), strip_spec, strip_spec,
                  pl.BlockSpec((1, 2 * _C1, _S), lambda bb, j: (bb, 0, 0))],
        out_specs=[pl.BlockSpec((1, 2 * _C1, _S), lambda bb, j: (bb, 0, 0)),
                   pl.BlockSpec((1, _S, 1), lambda bb, j: (bb, 0, 0))],
        out_shape=[jax.ShapeDtypeStruct((b, 2 * _C1, _S), jnp.float32),
                   jax.ShapeDtypeStruct((b, _S, 1), jnp.float32)],
        compiler_params=params,
    )(icol, xr, xkr, cent)

    out = pl.pallas_call(
        _iter2_kernel,
        grid=grid,
        in_specs=[whole((1, _STRIP)), strip_spec, strip_spec,
                  pl.BlockSpec((1, 2 * _C1, _S), lambda bb, j: (bb, 0, 0)),
                  pl.BlockSpec((1, _S, 1), lambda bb, j: (bb, 0, 0))],
        out_specs=pl.BlockSpec((1, _S, 2 * _STRIP), lambda bb, j: (bb, 0, j // 2)),
        out_shape=jax.ShapeDtypeStruct((b, _S, _W * _W), jnp.float32),
        scratch_shapes=[pltpu.VMEM((_S, _STRIP), jnp.float32)],
        compiler_params=params,
    )(icol, xr, xkr, num, den)

    return out, _S


# speedup vs baseline: 68.6137x; 1.0366x over previous
import jax
import jax.numpy as jnp
from jax import lax
from jax.experimental import pallas as pl
from jax.experimental.pallas import tpu as pltpu

_ST = 16
_NS = 14
_S = _NS * _NS
_C1 = 96
_W = 224
_STRIP = _ST * _W
_NSTRIP = _W // _ST

_MM = dict(preferred_element_type=jnp.float32)


def _split(a):
    ah = a.astype(jnp.bfloat16)
    al = (a - ah.astype(jnp.float32)).astype(jnp.bfloat16)
    return ah, al


def _dot3(a, b, dims):
    ah, al = _split(a)
    bh, bl = _split(b)
    d = lax.dot_general(ah, bh, (dims, ((), ())), **_MM)
    d += lax.dot_general(ah, bl, (dims, ((), ())), **_MM)
    d += lax.dot_general(al, bh, (dims, ((), ())), **_MM)
    return d


def _dot3p(ah, al, bh, bl, dims):
    d = lax.dot_general(ah, bh, (dims, ((), ())), **_MM)
    d += lax.dot_general(ah, bl, (dims, ((), ())), **_MM)
    d += lax.dot_general(al, bh, (dims, ((), ())), **_MM)
    return d


def _dot_e(a, b_exact, dims):
    ah, al = _split(a)
    be = b_exact.astype(jnp.bfloat16)
    d = lax.dot_general(ah, be, (dims, ((), ())), **_MM)
    d += lax.dot_general(al, be, (dims, ((), ())), **_MM)
    return d


def _dot_e2(a_exact, b, dims):
    ae = a_exact.astype(jnp.bfloat16)
    bh, bl = _split(b)
    d = lax.dot_general(ae, bh, (dims, ((), ())), **_MM)
    d += lax.dot_general(ae, bl, (dims, ((), ())), **_MM)
    return d


_WIN = 48


def _window(j):
    q = jnp.clip(j - 1, 0, 11)
    w0 = q * _NS
    s_iota = lax.broadcasted_iota(jnp.int32, (_S, _WIN), 0)
    r_iota = lax.broadcasted_iota(jnp.int32, (_S, _WIN), 1)
    pjt = (s_iota == r_iota + w0).astype(jnp.float32)
    s_iota2 = lax.broadcasted_iota(jnp.int32, (_WIN, _S), 1)
    r_iota2 = lax.broadcasted_iota(jnp.int32, (_WIN, _S), 0)
    pj = (s_iota2 == r_iota2 + w0).astype(jnp.float32)
    rcol = lax.broadcasted_iota(jnp.int32, (_WIN, 1), 0)
    rband = (rcol >= _NS).astype(jnp.int32) + (rcol >= 2 * _NS).astype(
        jnp.int32) + (rcol >= 3 * _NS).astype(jnp.int32)
    band_ok = (jnp.abs(rband + q - j) <= 1) & (rcol + w0 <= _S - 1)
    rmod = rcol - rband * _NS
    return pjt, pj, band_ok, rmod


def _affinities_w(icol_ref, pixs, centwc, centwk, band_ok, rmod):
    (pch, pcl), (pkh, pkl) = pixs
    d = _dot3p(*_split(centwc), pch, pcl, ((0,), (0,)))
    d += _dot3p(*_split(centwk * 10.0), pkh, pkl, ((0,), (0,)))
    c2 = (jnp.sum(centwc * centwc, axis=0, keepdims=True)
          + 10.0 * jnp.sum(centwk * centwk, axis=0, keepdims=True))
    dist = _dot_e(c2, jnp.ones((1, _STRIP), jnp.float32),
                  ((0,), (0,))) - 2.0 * d
    maskb = (jnp.abs(rmod - icol_ref[...]) <= 1) & band_ok
    mmin = jnp.min(jnp.where(maskb, dist, 1e30), axis=0, keepdims=True)
    z = jnp.where(maskb, mmin - dist, -1e9)
    e = jnp.exp(z)
    return e / jnp.sum(e, axis=0, keepdims=True)


def _pool_kernel(mpool_ref, xc_ref, xk_ref, cent_ref):
    j = pl.program_id(1)
    mp = mpool_ref[...]
    cellc = _dot_e(xc_ref[0], mp, ((1,), (0,))) * (1.0 / 256.0)
    cellk = _dot_e(xk_ref[0], mp, ((1,), (0,))) * (1.0 / 256.0)
    trow = lax.broadcasted_iota(jnp.int32, (_NS, _S), 0)
    scol = lax.broadcasted_iota(jnp.int32, (_NS, _S), 1)
    ej = (scol == trow + j * _NS).astype(jnp.float32)
    fc = _dot_e(cellc, ej, ((1,), (0,)))
    fk = _dot_e(cellk, ej, ((1,), (0,)))

    @pl.when(j == 0)
    def _init():
        cent_ref[0, :_C1, :] = fc
        cent_ref[0, _C1:, :] = fk

    @pl.when(j != 0)
    def _acc():
        cent_ref[0, :_C1, :] += fc
        cent_ref[0, _C1:, :] += fk


def _iter1_kernel(icol_ref, xc_ref, xk_ref, cent_ref, num_ref, den_ref):
    j = pl.program_id(1)
    pjt, pj, band_ok, rmod = _window(j)
    pixs = (_split(xc_ref[0]), _split(xk_ref[0]))
    centwc = _dot_e(cent_ref[0, :_C1, :], pjt, ((1,), (0,)))
    centwk = _dot_e(cent_ref[0, _C1:, :], pjt, ((1,), (0,)))
    aff = _affinities_w(icol_ref, pixs, centwc, centwk, band_ok, rmod)
    ah, al = _split(aff)
    afth, aftl = ah.T, al.T
    (pch, pcl), (pkh, pkl) = pixs
    ncw = _dot3p(pch, pcl, afth, aftl, ((1,), (0,)))
    nkw = _dot3p(pkh, pkl, afth, aftl, ((1,), (0,)))
    ddenw = jnp.sum(aff, axis=1, keepdims=True)
    nc = _dot_e(ncw, pj, ((1,), (0,)))
    nk = _dot_e(nkw, pj, ((1,), (0,)))
    dden = _dot_e2(pjt, ddenw, ((1,), (0,)))

    @pl.when(j == 0)
    def _init():
        num_ref[0, :_C1, :] = nc
        num_ref[0, _C1:, :] = nk
        den_ref[0] = dden

    @pl.when(j != 0)
    def _acc():
        num_ref[0, :_C1, :] += nc
        num_ref[0, _C1:, :] += nk
        den_ref[0] += dden


def _iter2_kernel(icol_ref, xc_ref, xk_ref, num_ref, den_ref, out_ref):
    j = pl.program_id(1)
    pjt, pj, band_ok, rmod = _window(j)
    numwc = _dot_e(num_ref[0, :_C1, :], pjt, ((1,), (0,)))
    numwk = _dot_e(num_ref[0, _C1:, :], pjt, ((1,), (0,)))
    denw = _dot_e2(pj, den_ref[0], ((1,), (0,)))
    r = (1.0 / (denw + 1e-16)).T
    pixs = (_split(xc_ref[0]), _split(xk_ref[0]))
    aff = _affinities_w(icol_ref, pixs, numwc * r, numwk * r, band_ok, rmod)
    out_ref[0] = _dot_e2(pjt, aff, ((1,), (0,)))


def kernel(x, x_contrast, stoken_size):
    del stoken_size
    b = x.shape[0]
    xr = x.reshape(b, _C1, _W * _W)
    xkr = x_contrast.reshape(b, _C1, _W * _W)

    icol = ((jnp.arange(_STRIP, dtype=jnp.int32) % _W) // _ST)[None, :]
    mpool = (((jnp.arange(_STRIP, dtype=jnp.int32) % _W) // _ST)[:, None]
             == jnp.arange(_NS, dtype=jnp.int32)[None, :]
             ).astype(jnp.float32)

    grid = (b, _NSTRIP)
    strip_spec = pl.BlockSpec((1, _C1, _STRIP), lambda bb, j: (bb, 0, j))
    whole = lambda shape: pl.BlockSpec(shape, lambda bb, j: (0,) * len(shape))
    params = pltpu.CompilerParams(
        dimension_semantics=("arbitrary", "arbitrary"))

    cent = pl.pallas_call(
        _pool_kernel,
        grid=grid,
        in_specs=[whole((_STRIP, _NS)), strip_spec, strip_spec],
        out_specs=pl.BlockSpec((1, 2 * _C1, _S), lambda bb, j: (bb, 0, 0)),
        out_shape=jax.ShapeDtypeStruct((b, 2 * _C1, _S), jnp.float32),
        compiler_params=params,
    )(mpool, xr, xkr)

    num, den = pl.pallas_call(
        _iter1_kernel,
        grid=grid,
        in_specs=[whole((1, _STRIP)), strip_spec, strip_spec,
                  pl.BlockSpec((1, 2 * _C1, _S), lambda bb, j: (bb, 0, 0))],
        out_specs=[pl.BlockSpec((1, 2 * _C1, _S), lambda bb, j: (bb, 0, 0)),
                   pl.BlockSpec((1, _S, 1), lambda bb, j: (bb, 0, 0))],
        out_shape=[jax.ShapeDtypeStruct((b, 2 * _C1, _S), jnp.float32),
                   jax.ShapeDtypeStruct((b, _S, 1), jnp.float32)],
        compiler_params=params,
    )(icol, xr, xkr, cent)

    out = pl.pallas_call(
        _iter2_kernel,
        grid=grid,
        in_specs=[whole((1, _STRIP)), strip_spec, strip_spec,
                  pl.BlockSpec((1, 2 * _C1, _S), lambda bb, j: (bb, 0, 0)),
                  pl.BlockSpec((1, _S, 1), lambda bb, j: (bb, 0, 0))],
        out_specs=pl.BlockSpec((1, _S, _STRIP), lambda bb, j: (bb, 0, j)),
        out_shape=jax.ShapeDtypeStruct((b, _S, _W * _W), jnp.float32),
        compiler_params=params,
    )(icol, xr, xkr, num, den)

    return out, _S


# speedup vs baseline: 68.6561x; 1.0006x over previous
import jax
import jax.numpy as jnp
from jax import lax
from jax.experimental import pallas as pl
from jax.experimental.pallas import tpu as pltpu

_ST = 16
_NS = 14
_S = _NS * _NS
_C1 = 96
_W = 224
_STRIP = _ST * _W
_NSTRIP = _W // _ST

_MM = dict(preferred_element_type=jnp.float32)


def _split(a):
    ah = a.astype(jnp.bfloat16)
    al = (a - ah.astype(jnp.float32)).astype(jnp.bfloat16)
    return ah, al


def _dot3p(ah, al, bh, bl, dims):
    d = lax.dot_general(ah, bh, (dims, ((), ())), **_MM)
    d += lax.dot_general(ah, bl, (dims, ((), ())), **_MM)
    d += lax.dot_general(al, bh, (dims, ((), ())), **_MM)
    return d


def _dot_e(a, b_exact, dims):
    ah, al = _split(a)
    be = b_exact.astype(jnp.bfloat16)
    d = lax.dot_general(ah, be, (dims, ((), ())), **_MM)
    d += lax.dot_general(al, be, (dims, ((), ())), **_MM)
    return d


def _dot_e2(a_exact, b, dims):
    ae = a_exact.astype(jnp.bfloat16)
    bh, bl = _split(b)
    d = lax.dot_general(ae, bh, (dims, ((), ())), **_MM)
    d += lax.dot_general(ae, bl, (dims, ((), ())), **_MM)
    return d


_WIN = 48


def _window(j):
    q = jnp.clip(j - 1, 0, 11)
    w0 = q * _NS
    s_iota = lax.broadcasted_iota(jnp.int32, (_S, _WIN), 0)
    r_iota = lax.broadcasted_iota(jnp.int32, (_S, _WIN), 1)
    pjt = (s_iota == r_iota + w0).astype(jnp.float32)
    s_iota2 = lax.broadcasted_iota(jnp.int32, (_WIN, _S), 1)
    r_iota2 = lax.broadcasted_iota(jnp.int32, (_WIN, _S), 0)
    pj = (s_iota2 == r_iota2 + w0).astype(jnp.float32)
    rcol = lax.broadcasted_iota(jnp.int32, (_WIN, 1), 0)
    rband = (rcol >= _NS).astype(jnp.int32) + (rcol >= 2 * _NS).astype(
        jnp.int32) + (rcol >= 3 * _NS).astype(jnp.int32)
    band_ok = (jnp.abs(rband + q - j) <= 1) & (rcol + w0 <= _S - 1)
    rmod = rcol - rband * _NS
    return pjt, pj, band_ok, rmod


def _affinities_w(icol_ref, pixs, centwc, centwk, band_ok, rmod):
    (pch, pcl), (pkh, pkl) = pixs
    d = _dot3p(*_split(centwc), pch, pcl, ((0,), (0,)))
    d += _dot3p(*_split(centwk * 10.0), pkh, pkl, ((0,), (0,)))
    c2 = (jnp.sum(centwc * centwc, axis=0, keepdims=True)
          + 10.0 * jnp.sum(centwk * centwk, axis=0, keepdims=True))
    dist = _dot_e(c2, jnp.ones((1, _STRIP), jnp.float32),
                  ((0,), (0,))) - 2.0 * d
    maskb = (jnp.abs(rmod - icol_ref[...]) <= 1) & band_ok
    mmin = jnp.min(jnp.where(maskb, dist, 1e30), axis=0, keepdims=True)
    z = jnp.where(maskb, mmin - dist, -1e9)
    e = jnp.exp(z)
    return e / jnp.sum(e, axis=0, keepdims=True)


def _pool_kernel(mpool_ref, xc_ref, xk_ref, cent_ref):
    j = pl.program_id(1)
    mp = mpool_ref[...]
    cellc = _dot_e(xc_ref[0], mp, ((1,), (0,))) * (1.0 / 256.0)
    cellk = _dot_e(xk_ref[0], mp, ((1,), (0,))) * (1.0 / 256.0)
    trow = lax.broadcasted_iota(jnp.int32, (_NS, _S), 0)
    scol = lax.broadcasted_iota(jnp.int32, (_NS, _S), 1)
    ej = (scol == trow + j * _NS).astype(jnp.float32)
    fc = _dot_e(cellc, ej, ((1,), (0,)))
    fk = _dot_e(cellk, ej, ((1,), (0,)))

    @pl.when(j == 0)
    def _init():
        cent_ref[0, :_C1, :] = fc
        cent_ref[0, _C1:, :] = fk

    @pl.when(j != 0)
    def _acc():
        cent_ref[0, :_C1, :] += fc
        cent_ref[0, _C1:, :] += fk


def _iter1_kernel(icol_ref, xc_ref, xk_ref, cent_ref, num_ref, den_ref):
    j = pl.program_id(1)
    pjt, pj, band_ok, rmod = _window(j)
    pixs = (_split(xc_ref[0]), _split(xk_ref[0]))
    centwc = _dot_e(cent_ref[0, :_C1, :], pjt, ((1,), (0,)))
    centwk = _dot_e(cent_ref[0, _C1:, :], pjt, ((1,), (0,)))
    aff = _affinities_w(icol_ref, pixs, centwc, centwk, band_ok, rmod)
    ah, al = _split(aff)
    afth, aftl = ah.T, al.T
    (pch, pcl), (pkh, pkl) = pixs
    ncw = _dot3p(pch, pcl, afth, aftl, ((1,), (0,)))
    nkw = _dot3p(pkh, pkl, afth, aftl, ((1,), (0,)))
    ddenw = jnp.sum(aff, axis=1, keepdims=True)
    nc = _dot_e(ncw, pj, ((1,), (0,)))
    nk = _dot_e(nkw, pj, ((1,), (0,)))
    dden = _dot_e2(pjt, ddenw, ((1,), (0,)))

    @pl.when(j == 0)
    def _init():
        num_ref[0, :_C1, :] = nc
        num_ref[0, _C1:, :] = nk
        den_ref[0] = dden

    @pl.when(j != 0)
    def _acc():
        num_ref[0, :_C1, :] += nc
        num_ref[0, _C1:, :] += nk
        den_ref[0] += dden


def _iter2_kernel(icol_ref, xc_ref, xk_ref, num_ref, den_ref, out_ref):
    j = pl.program_id(1)
    pjt, pj, band_ok, rmod = _window(j)
    numwc = _dot_e(num_ref[0, :_C1, :], pjt, ((1,), (0,)))
    numwk = _dot_e(num_ref[0, _C1:, :], pjt, ((1,), (0,)))
    denw = _dot_e2(pj, den_ref[0], ((1,), (0,)))
    r = (1.0 / (denw + 1e-16)).T
    pixs = (_split(xc_ref[0]), _split(xk_ref[0]))
    aff = _affinities_w(icol_ref, pixs, numwc * r, numwk * r, band_ok, rmod)
    out_ref[0] = _dot_e2(pjt, aff, ((1,), (0,)))


def kernel(x, x_contrast, stoken_size):
    del stoken_size
    b = x.shape[0]
    xr = x.reshape(b, _C1, _W * _W)
    xkr = x_contrast.reshape(b, _C1, _W * _W)

    icol = ((jnp.arange(_STRIP, dtype=jnp.int32) % _W) // _ST)[None, :]
    mpool = (((jnp.arange(_STRIP, dtype=jnp.int32) % _W) // _ST)[:, None]
             == jnp.arange(_NS, dtype=jnp.int32)[None, :]
             ).astype(jnp.float32)

    grid = (b, _NSTRIP)
    strip_spec = pl.BlockSpec((1, _C1, _STRIP), lambda bb, j: (bb, 0, j))
    whole = lambda shape: pl.BlockSpec(shape, lambda bb, j: (0,) * len(shape))
    params = pltpu.CompilerParams(
        dimension_semantics=("arbitrary", "arbitrary"))

    cent = pl.pallas_call(
        _pool_kernel,
        grid=grid,
        in_specs=[whole((_STRIP, _NS)), strip_spec, strip_spec],
        out_specs=pl.BlockSpec((1, 2 * _C1, _S), lambda bb, j: (bb, 0, 0)),
        out_shape=jax.ShapeDtypeStruct((b, 2 * _C1, _S), jnp.float32),
        compiler_params=params,
    )(mpool, xr, xkr)

    num, den = pl.pallas_call(
        _iter1_kernel,
        grid=grid,
        in_specs=[whole((1, _STRIP)), strip_spec, strip_spec,
                  pl.BlockSpec((1, 2 * _C1, _S), lambda bb, j: (bb, 0, 0))],
        out_specs=[pl.BlockSpec((1, 2 * _C1, _S), lambda bb, j: (bb, 0, 0)),
                   pl.BlockSpec((1, _S, 1), lambda bb, j: (bb, 0, 0))],
        out_shape=[jax.ShapeDtypeStruct((b, 2 * _C1, _S), jnp.float32),
                   jax.ShapeDtypeStruct((b, _S, 1), jnp.float32)],
        compiler_params=params,
    )(icol, xr, xkr, cent)

    out = pl.pallas_call(
        _iter2_kernel,
        grid=grid,
        in_specs=[whole((1, _STRIP)), strip_spec, strip_spec,
                  pl.BlockSpec((1, 2 * _C1, _S), lambda bb, j: (bb, 0, 0)),
                  pl.BlockSpec((1, _S, 1), lambda bb, j: (bb, 0, 0))],
        out_specs=pl.BlockSpec((1, _S, _STRIP), lambda bb, j: (bb, 0, j)),
        out_shape=jax.ShapeDtypeStruct((b, _S, _W * _W), jnp.float32),
        compiler_params=params,
    )(icol, xr, xkr, num, den)

    return out, _S
